# baseline (speedup 1.0000x reference)
import jax
import jax.numpy as jnp
from jax.experimental import pallas as pl
from jax.experimental.pallas import tpu as pltpu


def _copy_body(x1_ref, x2_ref, o1_ref, o2_ref, sem1, sem2):
    c1 = pltpu.make_async_copy(x1_ref, o1_ref, sem1)
    c2 = pltpu.make_async_copy(x2_ref, o2_ref, sem2)
    c1.start()
    c2.start()
    c1.wait()
    c2.wait()


def kernel(x1, x2):
    out1, out2 = pl.pallas_call(
        _copy_body,
        out_shape=(
            jax.ShapeDtypeStruct(x1.shape, x1.dtype),
            jax.ShapeDtypeStruct(x2.shape, x2.dtype),
        ),
        in_specs=[
            pl.BlockSpec(memory_space=pltpu.MemorySpace.HBM),
            pl.BlockSpec(memory_space=pltpu.MemorySpace.HBM),
        ],
        out_specs=(
            pl.BlockSpec(memory_space=pltpu.MemorySpace.HBM),
            pl.BlockSpec(memory_space=pltpu.MemorySpace.HBM),
        ),
        scratch_shapes=[pltpu.SemaphoreType.DMA, pltpu.SemaphoreType.DMA],
    )(x1, x2)
    return (out1, out2)


# speedup vs baseline: 12.5656x; 12.5656x over previous
import jax
import jax.numpy as jnp
from jax.experimental import pallas as pl
from jax.experimental.pallas import tpu as pltpu

_ROWS_PER_BLOCK = 16


def _copy_body(x1_ref, x2_ref, o1_ref, o2_ref):
    o1_ref[...] = x1_ref[...]
    o2_ref[...] = x2_ref[...]


def kernel(x1, x2):
    N, C, H, W = x1.shape
    rows = N * C
    cols = H * W
    a = x1.reshape(rows, cols)
    b = x2.reshape(rows, cols)
    grid = (rows // _ROWS_PER_BLOCK,)
    spec = pl.BlockSpec((_ROWS_PER_BLOCK, cols), lambda i: (i, 0))
    out1, out2 = pl.pallas_call(
        _copy_body,
        grid=grid,
        out_shape=(
            jax.ShapeDtypeStruct((rows, cols), x1.dtype),
            jax.ShapeDtypeStruct((rows, cols), x2.dtype),
        ),
        in_specs=[spec, spec],
        out_specs=(spec, spec),
    )(a, b)
    return (out1.reshape(N, C, H, W), out2.reshape(N, C, H, W))


# speedup vs baseline: 48.8491x; 3.8875x over previous
import jax
import jax.numpy as jnp
from jax.experimental import pallas as pl
from jax.experimental.pallas import tpu as pltpu

_ROWS_PER_BLOCK = 16


def _copy_body(x1_ref, x2_ref, o1_ref, o2_ref):
    o1_ref[...] = x1_ref[...]
    o2_ref[...] = x2_ref[...]


def kernel(x1, x2):
    N, C, H, W = x1.shape
    rows = N * C
    a = x1.reshape(rows, H, W)
    b = x2.reshape(rows, H, W)
    grid = (rows // _ROWS_PER_BLOCK,)
    spec = pl.BlockSpec((_ROWS_PER_BLOCK, H, W), lambda i: (i, 0, 0))
    out1, out2 = pl.pallas_call(
        _copy_body,
        grid=grid,
        out_shape=(
            jax.ShapeDtypeStruct((rows, H, W), x1.dtype),
            jax.ShapeDtypeStruct((rows, H, W), x2.dtype),
        ),
        in_specs=[spec, spec],
        out_specs=(spec, spec),
    )(a, b)
    return (out1.reshape(N, C, H, W), out2.reshape(N, C, H, W))


# speedup vs baseline: 49.1657x; 1.0065x over previous
import jax
import jax.numpy as jnp
from jax.experimental import pallas as pl
from jax.experimental.pallas import tpu as pltpu

_ROWS_PER_BLOCK = 32


def _copy_body(x1_ref, x2_ref, o1_ref, o2_ref):
    o1_ref[...] = x1_ref[...]
    o2_ref[...] = x2_ref[...]


def kernel(x1, x2):
    N, C, H, W = x1.shape
    rows = N * C
    a = x1.reshape(rows, H, W)
    b = x2.reshape(rows, H, W)
    grid = (rows // _ROWS_PER_BLOCK,)
    spec = pl.BlockSpec((_ROWS_PER_BLOCK, H, W), lambda i: (i, 0, 0))
    out1, out2 = pl.pallas_call(
        _copy_body,
        grid=grid,
        out_shape=(
            jax.ShapeDtypeStruct((rows, H, W), x1.dtype),
            jax.ShapeDtypeStruct((rows, H, W), x2.dtype),
        ),
        in_specs=[spec, spec],
        out_specs=(spec, spec),
    )(a, b)
    return (out1.reshape(N, C, H, W), out2.reshape(N, C, H, W))


# speedup vs baseline: 49.1722x; 1.0001x over previous
import jax
import jax.numpy as jnp
from jax.experimental import pallas as pl
from jax.experimental.pallas import tpu as pltpu

_ROWS_PER_BLOCK = 32


def _copy_body(x1_ref, x2_ref, o1_ref, o2_ref):
    o1_ref[...] = x1_ref[...]
    o2_ref[...] = x2_ref[...]


def kernel(x1, x2):
    N, C, H, W = x1.shape
    rows = N * C
    a = x1.reshape(rows, H, W)
    b = x2.reshape(rows, H, W)
    grid = (rows // _ROWS_PER_BLOCK,)
    spec = pl.BlockSpec((_ROWS_PER_BLOCK, H, W), lambda i: (i, 0, 0))
    out1, out2 = pl.pallas_call(
        _copy_body,
        grid=grid,
        out_shape=(
            jax.ShapeDtypeStruct((rows, H, W), x1.dtype),
            jax.ShapeDtypeStruct((rows, H, W), x2.dtype),
        ),
        in_specs=[spec, spec],
        out_specs=(spec, spec),
        compiler_params=pltpu.CompilerParams(
            dimension_semantics=("parallel",),
        ),
    )(a, b)
    return (out1.reshape(N, C, H, W), out2.reshape(N, C, H, W))


# speedup vs baseline: 49.2145x; 1.0009x over previous
import jax
import jax.numpy as jnp
from jax.experimental import pallas as pl
from jax.experimental.pallas import tpu as pltpu

_ROWS_PER_BLOCK = 32


def _copy_body(x1_ref, x2_ref, o1_ref, o2_ref):
    o1_ref[...] = x1_ref[...]
    o2_ref[...] = x2_ref[...]


def kernel(x1, x2):
    N, C, H, W = x1.shape
    rows = N * C
    a = x1.reshape(rows, H, W)
    b = x2.reshape(rows, H, W)
    grid = (rows // _ROWS_PER_BLOCK,)
    spec = pl.BlockSpec((_ROWS_PER_BLOCK, H, W), lambda i: (i, 0, 0))
    out1, out2 = pl.pallas_call(
        _copy_body,
        grid=grid,
        out_shape=(
            jax.ShapeDtypeStruct((rows, H, W), x1.dtype),
            jax.ShapeDtypeStruct((rows, H, W), x2.dtype),
        ),
        in_specs=[spec, spec],
        out_specs=(spec, spec),
        compiler_params=pltpu.CompilerParams(
            dimension_semantics=("parallel",),
            vmem_limit_bytes=128 * 1024 * 1024,
        ),
    )(a, b)
    return (out1.reshape(N, C, H, W), out2.reshape(N, C, H, W))
